# baseline (speedup 1.0000x reference)
import functools

import jax
import jax.numpy as jnp
from jax import lax
from jax.experimental import pallas as pl
from jax.experimental.pallas import tpu as pltpu
from jax.experimental.pallas import tpu_sc as plsc

N = 10000
NPAD = 10008
D = 256
H = 8
DH = 32
E = 160000
NTILES = 16
BATCH = 16
NB = 640
NBLK = NB // 2
EPTP = NB * BATCH
NG = BATCH // 16
ROWPT = N // NTILES
ACCW = 144
SCALE = 1.0 / (DH ** 0.5)


def _qkv_body(x_ref, wq_ref, wk_ref, wv_ref, qs, kvs):
    x = x_ref[...]
    dn = (((1,), (1,)), ((), ()))
    q = lax.dot_general(x, wq_ref[...], dn, preferred_element_type=jnp.float32)
    k = lax.dot_general(x, wk_ref[...], dn, preferred_element_type=jnp.float32)
    v = lax.dot_general(x, wv_ref[...], dn, preferred_element_type=jnp.float32)
    q = q * SCALE
    qs[0] = q[:, :128]
    qs[1] = q[:, 128:]
    kvs[0] = jnp.concatenate([k[:, :128], v[:, :128]], axis=1)
    kvs[1] = jnp.concatenate([k[:, 128:], v[:, 128:]], axis=1)


def _qkv(x, Wq, Wk, Wv):
    blk = 1000
    grid = (N // blk,)
    return pl.pallas_call(
        _qkv_body,
        grid=grid,
        in_specs=[
            pl.BlockSpec((blk, D), lambda i: (i, 0)),
            pl.BlockSpec((D, D), lambda i: (0, 0)),
            pl.BlockSpec((D, D), lambda i: (0, 0)),
            pl.BlockSpec((D, D), lambda i: (0, 0)),
        ],
        out_specs=[
            pl.BlockSpec((2, blk, 128), lambda i: (0, i, 0)),
            pl.BlockSpec((2, blk, 256), lambda i: (0, i, 0)),
        ],
        out_shape=[
            jax.ShapeDtypeStruct((2, N, 128), jnp.float32),
            jax.ShapeDtypeStruct((2, N, 256), jnp.float32),
        ],
    )(x, Wq, Wk, Wv)


def _sc_edge_call(qs, kvs, edata, we_exp):
    mesh = plsc.VectorSubcoreMesh(core_axis_name="c", subcore_axis_name="s")

    vb = lambda shape: pltpu.VMEM(shape, jnp.float32)
    ib = lambda shape: pltpu.VMEM(shape, jnp.int32)

    @functools.partial(
        pl.kernel,
        mesh=mesh,
        compiler_params=pltpu.CompilerParams(
            use_tc_tiling_on_sc=False, needs_layout_passes=False),
        out_type=jax.ShapeDtypeStruct((2, N, ACCW), jnp.float32),
        scratch_types=[
            ib((2, 4, BATCH)), ib((2, 4, BATCH)),
            ib((2, BATCH)), ib((2, BATCH)),
            vb((BATCH, 128)), vb((BATCH, 128)),
            vb((BATCH, 256)), vb((BATCH, 256)),
            vb((BATCH, ACCW)), vb((BATCH, ACCW)),
            vb((4, 16)),
            pltpu.VMEM_SHARED((NPAD, ACCW), jnp.float32),
        ] + [pltpu.SemaphoreType.DMA] * 8,
    )
    def sc_kernel(qs_h, kvs_h, ed_h, we_h, zn_h,
                  outn_h,
                  edidxA, edidxB, sidxA, sidxB, qvA, qvB, kvA, kvB,
                  vvA, vvB, webuf, accn,
                  semIA, semIB, semGqA, semGqB, semGkA, semGkB,
                  semSnA, semSnB):
        cid = lax.axis_index("c")
        sid = lax.axis_index("s")

        edidx = [edidxA, edidxB]
        sidx = [sidxA, sidxB]
        qv = [qvA, qvB]
        kvb = [kvA, kvB]
        vvst = [vvA, vvB]
        semI = [semIA, semIB]
        semGq = [semGqA, semGqB]
        semGk = [semGkA, semGkB]
        semSn = [semSnA, semSnB]

        pltpu.sync_copy(we_h.at[cid], webuf)

        zero = jnp.zeros((16,), jnp.float32)

        pltpu.sync_copy(zn_h.at[pl.ds(0, ROWPT)],
                        accn.at[pl.ds(sid * ROWPT, ROWPT)])

        @pl.when(sid == NTILES - 1)
        def _():
            pltpu.sync_copy(zn_h.at[pl.ds(0, NPAD - N)], accn.at[pl.ds(N, NPAD - N)])

        plsc.subcore_barrier()

        def run_half(qh, kvh):
            iota16 = lax.iota(jnp.int32, 16)
            wrows = [webuf[hh, pl.ds(0, 16)] for hh in range(4)]
            c15 = jnp.full((16,), 15, jnp.int32)
            hmask = [iota16 == jnp.full((16,), hh, jnp.int32) for hh in range(4)]
            gdn = lax.GatherDimensionNumbers(
                offset_dims=(), collapsed_slice_dims=(0,), start_index_map=(0,))

            def bcast(x, idx):
                return lax.gather(
                    x, idx[:, None], gdn, (1,),
                    mode=lax.GatherScatterMode.PROMISE_IN_BOUNDS)

            def snapshot(p, b, q4):
                sidx[p][0, pl.ds(0, 16)] = edidx[b][q4, 3, pl.ds(0, 16)]
                sidx[p][1, pl.ds(0, 16)] = edidx[b][q4, 2, pl.ds(0, 16)]

            def compute(p):
                ewvec = plsc.bitcast(sidx[p][1, pl.ds(0, 16)], jnp.float32)
                for e in range(BATCH):
                    ewe = bcast(ewvec, jnp.full((16,), e, jnp.int32))
                    dvec = zero
                    for h in range(4):
                        q0 = qv[p][e, pl.ds(32 * h, 16)]
                        q1 = qv[p][e, pl.ds(32 * h + 16, 16)]
                        k0 = kvb[p][e, pl.ds(32 * h, 16)]
                        k1 = kvb[p][e, pl.ds(32 * h + 16, 16)]
                        cs = plsc.cumsum(q0 * k0 + q1 * k1)
                        tot = bcast(cs, c15)
                        ph = jnp.exp(tot + ewe * wrows[h])
                        v0 = kvb[p][e, pl.ds(128 + 32 * h, 16)]
                        v1 = kvb[p][e, pl.ds(128 + 32 * h + 16, 16)]
                        vvst[p][e, pl.ds(32 * h, 16)] = v0 * ph
                        vvst[p][e, pl.ds(32 * h + 16, 16)] = v1 * ph
                        dvec = jnp.where(hmask[h], ph, dvec)
                    vvst[p][e, pl.ds(128, 16)] = dvec

            pltpu.sync_copy(ed_h.at[sid, 0], edidxA)
            pltpu.sync_copy(ed_h.at[sid, 1], edidxB)
            pltpu.async_copy(qh.at[edidxA.at[0, 0]], qvA, semGqA)
            pltpu.async_copy(kvh.at[edidxA.at[0, 1]], kvA, semGkA)

            def body(i, carry):
                for k in range(4):
                    p = k % 2
                    b = k // 2
                    q4 = k % 2
                    pltpu.make_async_copy(
                        qh.at[edidx[b].at[q4, 0]], qv[p], semGq[p]).wait()
                    pltpu.make_async_copy(
                        kvh.at[edidx[b].at[q4, 1]], kvb[p], semGk[p]).wait()

                    if k < 2:
                        @pl.when(i > 0)
                        def _():
                            pltpu.make_async_copy(
                                outn_h.at[cid, pl.ds(0, BATCH)], vvst[p], semSn[p]).wait()
                    else:
                        pltpu.make_async_copy(
                            outn_h.at[cid, pl.ds(0, BATCH)], vvst[p], semSn[p]).wait()

                    snapshot(p, b, q4)

                    if k == 1:
                        bn = jnp.minimum(2 * i + 2, NBLK - 1)
                        pltpu.async_copy(ed_h.at[sid, bn], edidx[0], semI[0])

                        @pl.when(i > 0)
                        def _():
                            pltpu.make_async_copy(
                                ed_h.at[sid, 0], edidx[1], semI[1]).wait()
                    if k == 3:
                        bn = jnp.minimum(2 * i + 3, NBLK - 1)
                        pltpu.async_copy(ed_h.at[sid, bn], edidx[1], semI[1])
                        pltpu.make_async_copy(
                            ed_h.at[sid, 0], edidx[0], semI[0]).wait()

                    bn2 = ((k + 1) // 2) % 2
                    qn = (k + 1) % 2
                    pn = (k + 1) % 2
                    if k == 3:
                        @pl.when(i < NB // 4 - 1)
                        def _():
                            pltpu.async_copy(qh.at[edidx[0].at[0, 0]], qv[0], semGq[0])
                            pltpu.async_copy(kvh.at[edidx[0].at[0, 1]], kvb[0], semGk[0])
                    else:
                        pltpu.async_copy(qh.at[edidx[bn2].at[qn, 0]], qv[pn], semGq[pn])
                        pltpu.async_copy(kvh.at[edidx[bn2].at[qn, 1]], kvb[pn], semGk[pn])

                    compute(p)
                    pltpu.async_copy(vvst[p], accn.at[sidx[p].at[0]], semSn[p], add=True)
                return carry

            lax.fori_loop(0, NB // 4, body, 0)

            for p in range(2):
                pltpu.make_async_copy(
                    outn_h.at[cid, pl.ds(0, BATCH)], vvst[p], semSn[p]).wait()
            pltpu.make_async_copy(ed_h.at[sid, 0], edidxB, semIB).wait()

        run_half(qs_h.at[cid], kvs_h.at[cid])

        plsc.subcore_barrier()
        pltpu.sync_copy(accn.at[pl.ds(sid * ROWPT, ROWPT)],
                        outn_h.at[cid, pl.ds(sid * ROWPT, ROWPT)])

    zn = jnp.zeros((ROWPT + 8, ACCW), jnp.float32)
    return sc_kernel(qs, kvs, edata, we_exp, zn)


def _combine_body(a0_ref, a1_ref, x_ref, wo_ref, g_ref, b_ref, s_ref, o_ref):
    a0 = a0_ref[...]
    a1 = a1_ref[...]
    S = s_ref[...]
    dn = (((1,), (0,)), ((), ()))
    de0 = lax.dot_general(a0[:, 128:132], S, dn, preferred_element_type=jnp.float32)
    de1 = lax.dot_general(a1[:, 128:132], S, dn, preferred_element_type=jnp.float32)
    de0 = jnp.where(de0 > 0.0, de0, 1.0)
    de1 = jnp.where(de1 > 0.0, de1, 1.0)
    attn = jnp.concatenate([a0[:, :128] / de0, a1[:, :128] / de1], axis=1)
    dnt = (((1,), (1,)), ((), ()))
    out = lax.dot_general(attn, wo_ref[...], dnt, preferred_element_type=jnp.float32)
    y = out + x_ref[...]
    mean = jnp.mean(y, axis=1, keepdims=True)
    var = jnp.mean((y - mean) ** 2, axis=1, keepdims=True)
    o_ref[...] = (y - mean) * lax.rsqrt(var + 1e-5) * g_ref[...] + b_ref[...]


def _combine(a0, a1, x, Wo, gamma, beta, S):
    blk = 1000
    grid = (N // blk,)
    return pl.pallas_call(
        _combine_body,
        grid=grid,
        in_specs=[
            pl.BlockSpec((blk, ACCW), lambda i: (i, 0)),
            pl.BlockSpec((blk, ACCW), lambda i: (i, 0)),
            pl.BlockSpec((blk, D), lambda i: (i, 0)),
            pl.BlockSpec((D, D), lambda i: (0, 0)),
            pl.BlockSpec((1, D), lambda i: (0, 0)),
            pl.BlockSpec((1, D), lambda i: (0, 0)),
            pl.BlockSpec((4, 128), lambda i: (0, 0)),
        ],
        out_specs=pl.BlockSpec((blk, D), lambda i: (i, 0)),
        out_shape=jax.ShapeDtypeStruct((N, D), jnp.float32),
    )(a0, a1, x, Wo, gamma, beta, S)


def kernel(node_embeddings, edge_index, edge_weights, Wq, Wk, Wv, We, Wo, gamma, beta):
    x = node_embeddings[0]
    qs, kvs = _qkv(x, Wq, Wk, Wv)

    pad = EPTP - E // NTILES
    ei = edge_index.astype(jnp.int32)
    src = ei[0].reshape(NTILES, E // NTILES)
    tgt = ei[1].reshape(NTILES, E // NTILES)
    ewb = lax.bitcast_convert_type(edge_weights, jnp.int32).reshape(NTILES, -1)
    zpad = jnp.zeros((NTILES, pad), jnp.int32)
    srcg = jnp.concatenate([src, zpad], 1).reshape(NTILES, NB, BATCH)
    tgtp = jnp.concatenate([tgt, zpad], 1).reshape(NTILES, NB, BATCH)
    ewbp = jnp.concatenate([ewb, zpad], 1).reshape(NTILES, NB, BATCH)
    srcs = jnp.concatenate([src, jnp.full((NTILES, pad), N, jnp.int32)], 1)
    srcs = srcs.reshape(NTILES, NB, BATCH)
    edata = jnp.stack([srcg, tgtp, ewbp, srcs], axis=2)
    edata = edata.reshape(NTILES, NBLK, 2, 4, BATCH)

    we_exp = jnp.broadcast_to(We[:, 0].reshape(2, 4, 1), (2, 4, 16)).astype(jnp.float32)

    outn = _sc_edge_call(qs, kvs, edata, we_exp)

    S = (jnp.arange(128)[None, :] // DH == jnp.arange(4)[:, None]).astype(jnp.float32)
    ln = _combine(outn[0], outn[1], x, Wo,
                  gamma.reshape(1, D), beta.reshape(1, D), S)
    return ln[None]


# speedup vs baseline: 1.2142x; 1.2142x over previous
import functools

import jax
import jax.numpy as jnp
from jax import lax
from jax.experimental import pallas as pl
from jax.experimental.pallas import tpu as pltpu
from jax.experimental.pallas import tpu_sc as plsc

N = 10000
NPAD = 10008
D = 256
H = 8
DH = 32
E = 160000
NTILES = 16
BATCH = 32
NB = 316
NBLK = NB // 2
EPTP = NB * BATCH
NG = BATCH // 16
ROWPT = N // NTILES
ACCW = 144
SCALE = 1.0 / (DH ** 0.5)


def _qkv_body(x_ref, wq_ref, wk_ref, wv_ref, qs, kvs):
    x = x_ref[...]
    dn = (((1,), (1,)), ((), ()))
    q = lax.dot_general(x, wq_ref[...], dn, preferred_element_type=jnp.float32)
    k = lax.dot_general(x, wk_ref[...], dn, preferred_element_type=jnp.float32)
    v = lax.dot_general(x, wv_ref[...], dn, preferred_element_type=jnp.float32)
    q = q * SCALE
    qs[0] = q[:, :128]
    qs[1] = q[:, 128:]
    kvs[0] = jnp.concatenate([k[:, :128], v[:, :128]], axis=1)
    kvs[1] = jnp.concatenate([k[:, 128:], v[:, 128:]], axis=1)


def _qkv(x, Wq, Wk, Wv):
    blk = 1000
    grid = (N // blk,)
    return pl.pallas_call(
        _qkv_body,
        grid=grid,
        in_specs=[
            pl.BlockSpec((blk, D), lambda i: (i, 0)),
            pl.BlockSpec((D, D), lambda i: (0, 0)),
            pl.BlockSpec((D, D), lambda i: (0, 0)),
            pl.BlockSpec((D, D), lambda i: (0, 0)),
        ],
        out_specs=[
            pl.BlockSpec((2, blk, 128), lambda i: (0, i, 0)),
            pl.BlockSpec((2, blk, 256), lambda i: (0, i, 0)),
        ],
        out_shape=[
            jax.ShapeDtypeStruct((2, N, 128), jnp.float32),
            jax.ShapeDtypeStruct((2, N, 256), jnp.float32),
        ],
    )(x, Wq, Wk, Wv)


def _sc_edge_call(qs, kvs, edata, we_exp):
    mesh = plsc.VectorSubcoreMesh(core_axis_name="c", subcore_axis_name="s")

    vb = lambda shape: pltpu.VMEM(shape, jnp.float32)
    ib = lambda shape: pltpu.VMEM(shape, jnp.int32)

    @functools.partial(
        pl.kernel,
        mesh=mesh,
        compiler_params=pltpu.CompilerParams(
            use_tc_tiling_on_sc=False, needs_layout_passes=False),
        out_type=jax.ShapeDtypeStruct((2, N, ACCW), jnp.float32),
        scratch_types=[
            ib((2, 4, BATCH)), ib((2, 4, BATCH)),
            ib((2, BATCH)), ib((2, BATCH)),
            vb((BATCH, 128)), vb((BATCH, 128)),
            vb((BATCH, 256)), vb((BATCH, 256)),
            vb((BATCH, ACCW)), vb((BATCH, ACCW)),
            vb((4, 16)),
            pltpu.VMEM_SHARED((NPAD, ACCW), jnp.float32),
        ] + [pltpu.SemaphoreType.DMA] * 8,
    )
    def sc_kernel(qs_h, kvs_h, ed_h, we_h, zn_h,
                  outn_h,
                  edidxA, edidxB, sidxA, sidxB, qvA, qvB, kvA, kvB,
                  vvA, vvB, webuf, accn,
                  semIA, semIB, semGqA, semGqB, semGkA, semGkB,
                  semSnA, semSnB):
        cid = lax.axis_index("c")
        sid = lax.axis_index("s")

        edidx = [edidxA, edidxB]
        sidx = [sidxA, sidxB]
        qv = [qvA, qvB]
        kvb = [kvA, kvB]
        vvst = [vvA, vvB]
        semI = [semIA, semIB]
        semGq = [semGqA, semGqB]
        semGk = [semGkA, semGkB]
        semSn = [semSnA, semSnB]

        pltpu.sync_copy(we_h.at[cid], webuf)

        zero = jnp.zeros((16,), jnp.float32)

        pltpu.sync_copy(zn_h.at[pl.ds(0, ROWPT)],
                        accn.at[pl.ds(sid * ROWPT, ROWPT)])

        @pl.when(sid == NTILES - 1)
        def _():
            pltpu.sync_copy(zn_h.at[pl.ds(0, NPAD - N)], accn.at[pl.ds(N, NPAD - N)])

        plsc.subcore_barrier()

        def run_half(qh, kvh):
            iota16 = lax.iota(jnp.int32, 16)
            wrows = [webuf[hh, pl.ds(0, 16)] for hh in range(4)]
            c15 = jnp.full((16,), 15, jnp.int32)
            hmask = [iota16 == jnp.full((16,), hh, jnp.int32) for hh in range(4)]
            gdn = lax.GatherDimensionNumbers(
                offset_dims=(), collapsed_slice_dims=(0,), start_index_map=(0,))

            def bcast(x, idx):
                return lax.gather(
                    x, idx[:, None], gdn, (1,),
                    mode=lax.GatherScatterMode.PROMISE_IN_BOUNDS)

            def snapshot(p, b, q4):
                for g in range(NG):
                    sidx[p][0, pl.ds(16 * g, 16)] = edidx[b][q4, 3, pl.ds(16 * g, 16)]
                    sidx[p][1, pl.ds(16 * g, 16)] = edidx[b][q4, 2, pl.ds(16 * g, 16)]

            def compute(p):
                ewvec = [plsc.bitcast(sidx[p][1, pl.ds(16 * g, 16)], jnp.float32)
                         for g in range(NG)]
                for e in range(BATCH):
                    ewe = bcast(ewvec[e // 16], jnp.full((16,), e % 16, jnp.int32))
                    dvec = zero
                    for h in range(4):
                        q0 = qv[p][e, pl.ds(32 * h, 16)]
                        q1 = qv[p][e, pl.ds(32 * h + 16, 16)]
                        k0 = kvb[p][e, pl.ds(32 * h, 16)]
                        k1 = kvb[p][e, pl.ds(32 * h + 16, 16)]
                        cs = plsc.cumsum(q0 * k0 + q1 * k1)
                        tot = bcast(cs, c15)
                        ph = jnp.exp(tot + ewe * wrows[h])
                        v0 = kvb[p][e, pl.ds(128 + 32 * h, 16)]
                        v1 = kvb[p][e, pl.ds(128 + 32 * h + 16, 16)]
                        vvst[p][e, pl.ds(32 * h, 16)] = v0 * ph
                        vvst[p][e, pl.ds(32 * h + 16, 16)] = v1 * ph
                        dvec = jnp.where(hmask[h], ph, dvec)
                    vvst[p][e, pl.ds(128, 16)] = dvec

            pltpu.sync_copy(ed_h.at[sid, 0], edidxA)
            pltpu.sync_copy(ed_h.at[sid, 1], edidxB)
            pltpu.async_copy(qh.at[edidxA.at[0, 0]], qvA, semGqA)
            pltpu.async_copy(kvh.at[edidxA.at[0, 1]], kvA, semGkA)

            def body(i, carry):
                for k in range(4):
                    p = k % 2
                    b = k // 2
                    q4 = k % 2
                    pltpu.make_async_copy(
                        qh.at[edidx[b].at[q4, 0]], qv[p], semGq[p]).wait()
                    pltpu.make_async_copy(
                        kvh.at[edidx[b].at[q4, 1]], kvb[p], semGk[p]).wait()

                    if k < 2:
                        @pl.when(i > 0)
                        def _():
                            pltpu.make_async_copy(
                                outn_h.at[cid, pl.ds(0, BATCH)], vvst[p], semSn[p]).wait()
                    else:
                        pltpu.make_async_copy(
                            outn_h.at[cid, pl.ds(0, BATCH)], vvst[p], semSn[p]).wait()

                    snapshot(p, b, q4)

                    if k == 1:
                        bn = jnp.minimum(2 * i + 2, NBLK - 1)
                        pltpu.async_copy(ed_h.at[sid, bn], edidx[0], semI[0])

                        @pl.when(i > 0)
                        def _():
                            pltpu.make_async_copy(
                                ed_h.at[sid, 0], edidx[1], semI[1]).wait()
                    if k == 3:
                        bn = jnp.minimum(2 * i + 3, NBLK - 1)
                        pltpu.async_copy(ed_h.at[sid, bn], edidx[1], semI[1])
                        pltpu.make_async_copy(
                            ed_h.at[sid, 0], edidx[0], semI[0]).wait()

                    bn2 = ((k + 1) // 2) % 2
                    qn = (k + 1) % 2
                    pn = (k + 1) % 2
                    if k == 3:
                        @pl.when(i < NB // 4 - 1)
                        def _():
                            pltpu.async_copy(qh.at[edidx[0].at[0, 0]], qv[0], semGq[0])
                            pltpu.async_copy(kvh.at[edidx[0].at[0, 1]], kvb[0], semGk[0])
                    else:
                        pltpu.async_copy(qh.at[edidx[bn2].at[qn, 0]], qv[pn], semGq[pn])
                        pltpu.async_copy(kvh.at[edidx[bn2].at[qn, 1]], kvb[pn], semGk[pn])

                    compute(p)
                    pltpu.async_copy(vvst[p], accn.at[sidx[p].at[0]], semSn[p], add=True)
                return carry

            lax.fori_loop(0, NB // 4, body, 0)

            for p in range(2):
                pltpu.make_async_copy(
                    outn_h.at[cid, pl.ds(0, BATCH)], vvst[p], semSn[p]).wait()
            pltpu.make_async_copy(ed_h.at[sid, 0], edidxB, semIB).wait()

        run_half(qs_h.at[cid], kvs_h.at[cid])

        plsc.subcore_barrier()
        pltpu.sync_copy(accn.at[pl.ds(sid * ROWPT, ROWPT)],
                        outn_h.at[cid, pl.ds(sid * ROWPT, ROWPT)])

    zn = jnp.zeros((ROWPT + 8, ACCW), jnp.float32)
    return sc_kernel(qs, kvs, edata, we_exp, zn)


def _combine_body(a0_ref, a1_ref, x_ref, wo_ref, g_ref, b_ref, s_ref, o_ref):
    a0 = a0_ref[...]
    a1 = a1_ref[...]
    S = s_ref[...]
    dn = (((1,), (0,)), ((), ()))
    de0 = lax.dot_general(a0[:, 128:132], S, dn, preferred_element_type=jnp.float32)
    de1 = lax.dot_general(a1[:, 128:132], S, dn, preferred_element_type=jnp.float32)
    de0 = jnp.where(de0 > 0.0, de0, 1.0)
    de1 = jnp.where(de1 > 0.0, de1, 1.0)
    attn = jnp.concatenate([a0[:, :128] / de0, a1[:, :128] / de1], axis=1)
    dnt = (((1,), (1,)), ((), ()))
    out = lax.dot_general(attn, wo_ref[...], dnt, preferred_element_type=jnp.float32)
    y = out + x_ref[...]
    mean = jnp.mean(y, axis=1, keepdims=True)
    var = jnp.mean((y - mean) ** 2, axis=1, keepdims=True)
    o_ref[...] = (y - mean) * lax.rsqrt(var + 1e-5) * g_ref[...] + b_ref[...]


def _combine(a0, a1, x, Wo, gamma, beta, S):
    blk = 1000
    grid = (N // blk,)
    return pl.pallas_call(
        _combine_body,
        grid=grid,
        in_specs=[
            pl.BlockSpec((blk, ACCW), lambda i: (i, 0)),
            pl.BlockSpec((blk, ACCW), lambda i: (i, 0)),
            pl.BlockSpec((blk, D), lambda i: (i, 0)),
            pl.BlockSpec((D, D), lambda i: (0, 0)),
            pl.BlockSpec((1, D), lambda i: (0, 0)),
            pl.BlockSpec((1, D), lambda i: (0, 0)),
            pl.BlockSpec((4, 128), lambda i: (0, 0)),
        ],
        out_specs=pl.BlockSpec((blk, D), lambda i: (i, 0)),
        out_shape=jax.ShapeDtypeStruct((N, D), jnp.float32),
    )(a0, a1, x, Wo, gamma, beta, S)


def kernel(node_embeddings, edge_index, edge_weights, Wq, Wk, Wv, We, Wo, gamma, beta):
    x = node_embeddings[0]
    qs, kvs = _qkv(x, Wq, Wk, Wv)

    pad = EPTP - E // NTILES
    ei = edge_index.astype(jnp.int32)
    src = ei[0].reshape(NTILES, E // NTILES)
    tgt = ei[1].reshape(NTILES, E // NTILES)
    ewb = lax.bitcast_convert_type(edge_weights, jnp.int32).reshape(NTILES, -1)
    zpad = jnp.zeros((NTILES, pad), jnp.int32)
    srcg = jnp.concatenate([src, zpad], 1).reshape(NTILES, NB, BATCH)
    tgtp = jnp.concatenate([tgt, zpad], 1).reshape(NTILES, NB, BATCH)
    ewbp = jnp.concatenate([ewb, zpad], 1).reshape(NTILES, NB, BATCH)
    srcs = jnp.concatenate([src, jnp.full((NTILES, pad), N, jnp.int32)], 1)
    srcs = srcs.reshape(NTILES, NB, BATCH)
    edata = jnp.stack([srcg, tgtp, ewbp, srcs], axis=2)
    edata = edata.reshape(NTILES, NBLK, 2, 4, BATCH)

    we_exp = jnp.broadcast_to(We[:, 0].reshape(2, 4, 1), (2, 4, 16)).astype(jnp.float32)

    outn = _sc_edge_call(qs, kvs, edata, we_exp)

    S = (jnp.arange(128)[None, :] // DH == jnp.arange(4)[:, None]).astype(jnp.float32)
    ln = _combine(outn[0], outn[1], x, Wo,
                  gamma.reshape(1, D), beta.reshape(1, D), S)
    return ln[None]
